# Initial kernel scaffold; baseline (speedup 1.0000x reference)
import jax
import jax.numpy as jnp
from jax.experimental import pallas as pl


def kernel(inputs_ids, offsets, emb_weight, fc_weight, fc_bias):
    raise NotImplementedError("write your pallas kernel here")


# speedup vs baseline: 725.3434x; 725.3434x over previous
import functools

import jax
import jax.numpy as jnp
from jax import lax
from jax.experimental import pallas as pl
from jax.experimental.pallas import tpu as pltpu
from jax.experimental.pallas import tpu_sc as plsc

VOCAB = 100000
EMBED = 32
NUM_CLASS = 2
BATCH = 16384
TOTAL = 819200

NCORES = 2
NSUB = 16
LANES = 16

TOK_PER_SUB = TOTAL // NSUB
CHUNK = 5120
NCHUNKS = TOK_PER_SUB // CHUNK
GROUPS = CHUNK // LANES
OFF_PAD = BATCH + 2 * LANES
ACC_PAD = BATCH + LANES
BAGS_PER_SUB = BATCH // NSUB


def _proj_body(fc_ref, emb_ref, out_ref):
    out_ref[...] = lax.dot_general(
        fc_ref[...], emb_ref[...],
        dimension_numbers=(((1,), (1,)), ((), ())),
        preferred_element_type=jnp.float32,
    )


def _project(emb_weight, fc_weight):
    VB = 8192
    grid = (VOCAB + VB - 1) // VB
    return pl.pallas_call(
        _proj_body,
        grid=(grid,),
        in_specs=[
            pl.BlockSpec((NUM_CLASS, EMBED), lambda i: (0, 0)),
            pl.BlockSpec((VB, EMBED), lambda i: (i, 0)),
        ],
        out_specs=pl.BlockSpec((NUM_CLASS, VB), lambda i: (0, i)),
        out_shape=jax.ShapeDtypeStruct((NUM_CLASS, VOCAB), jnp.float32),
    )(fc_weight, emb_weight)


def _count_le(off_ref, p_vec):
    r = jnp.zeros((LANES,), jnp.int32)
    step = BATCH // 2
    while step >= 1:
        v = plsc.load_gather(off_ref, [r + (step - 1)])
        r = jnp.where(v <= p_vec, r + step, r)
        step //= 2
    return r


def _bag_body(ids_hbm, off_hbm, pt0_hbm, pt1_hbm, out_hbm,
              ids_v, vals_v, pinc_v, off_v, idx16_v, c16_v, bags_v, acc_sh,
              sem):
    cid = lax.axis_index("c")
    sid = lax.axis_index("s")

    pltpu.sync_copy(off_hbm, off_v.at[pl.ds(0, BATCH)])
    for j in range((OFF_PAD - BATCH) // LANES):
        off_v[pl.ds(BATCH + j * LANES, LANES)] = jnp.full(
            (LANES,), TOTAL, dtype=jnp.int32)

    @pl.when(sid == 0)
    def _zero():
        def zloop(g, _):
            pinc_v[pl.ds(g * LANES, LANES)] = jnp.zeros((LANES,), jnp.float32)
            return 0
        lax.fori_loop(0, GROUPS, zloop, 0)
        for piece in range(3):
            pltpu.sync_copy(pinc_v, acc_sh.at[pl.ds(piece * CHUNK, CHUNK)])
        pltpu.sync_copy(pinc_v.at[pl.ds(0, ACC_PAD - 3 * CHUNK)],
                        acc_sh.at[pl.ds(3 * CHUNK, ACC_PAD - 3 * CHUNK)])

    plsc.subcore_barrier()

    def chunk_body(ci, _):
        tok0 = sid * TOK_PER_SUB + ci * CHUNK

        pltpu.sync_copy(ids_hbm.at[pl.ds(tok0, CHUNK)], ids_v)

        @pl.when(cid == 0)
        def _g0():
            pltpu.async_copy(pt0_hbm.at[ids_v], vals_v, sem).wait()

        @pl.when(cid == 1)
        def _g1():
            pltpu.async_copy(pt1_hbm.at[ids_v], vals_v, sem).wait()

        def cs_body(g, carry):
            v = vals_v[pl.ds(g * LANES, LANES)]
            pinc_v[pl.ds(g * LANES, LANES)] = plsc.cumsum(v) + carry
            return carry + jnp.sum(v)
        lax.fori_loop(0, GROUPS, cs_body, jnp.float32(0.0))

        lane = lax.iota(jnp.int32, LANES)
        p_vec = jnp.where(lane == 0, tok0, tok0 + CHUNK - 1)
        r_vec = _count_le(off_v, p_vec) - 1
        b_lo = r_vec[0]
        b_hi = r_vec[1]
        ngroups = (b_hi - b_lo) // LANES + 1

        def bag_body(g, _):
            bv = b_lo + g * LANES + lax.iota(jnp.int32, LANES)
            s16 = plsc.load_gather(off_v, [bv])
            e16 = plsc.load_gather(off_v, [bv + 1])
            sc = jnp.clip(s16, tok0, tok0 + CHUNK)
            ec = jnp.clip(e16, tok0, tok0 + CHUNK)
            ilo = sc - tok0 - 1
            ihi = ec - tok0 - 1
            plo = jnp.where(
                ilo < 0, jnp.float32(0.0),
                plsc.load_gather(pinc_v, [jnp.maximum(ilo, 0)]))
            phi = jnp.where(
                ihi < 0, jnp.float32(0.0),
                plsc.load_gather(pinc_v, [jnp.maximum(ihi, 0)]))
            idx16_v[...] = bv
            c16_v[...] = phi - plo
            pltpu.sync_copy(c16_v, acc_sh.at[idx16_v], add=True)
            return 0
        lax.fori_loop(0, ngroups, bag_body, 0)
        return 0

    lax.fori_loop(0, NCHUNKS, chunk_body, 0)

    plsc.subcore_barrier()

    b0 = sid * BAGS_PER_SUB
    pltpu.sync_copy(acc_sh.at[pl.ds(b0, BAGS_PER_SUB)], bags_v)

    def fin_body(g, _):
        bv = b0 + g * LANES + lax.iota(jnp.int32, LANES)
        s16 = plsc.load_gather(off_v, [bv])
        e16 = plsc.load_gather(off_v, [bv + 1])
        cnt = jnp.maximum((e16 - s16).astype(jnp.float32), 1.0)
        bags_v[pl.ds(g * LANES, LANES)] = (
            bags_v[pl.ds(g * LANES, LANES)] / cnt)
        return 0
    lax.fori_loop(0, BAGS_PER_SUB // LANES, fin_body, 0)

    @pl.when(cid == 0)
    def _w0():
        pltpu.sync_copy(bags_v, out_hbm.at[0, pl.ds(b0, BAGS_PER_SUB)])

    @pl.when(cid == 1)
    def _w1():
        pltpu.sync_copy(bags_v, out_hbm.at[1, pl.ds(b0, BAGS_PER_SUB)])


def _bag_means(ids, offsets, pt0, pt1):
    mesh = plsc.VectorSubcoreMesh(core_axis_name="c", subcore_axis_name="s")
    return pl.kernel(
        _bag_body,
        out_type=jax.ShapeDtypeStruct((NUM_CLASS, BATCH), jnp.float32),
        mesh=mesh,
        scratch_types=[
            pltpu.VMEM((CHUNK,), jnp.int32),
            pltpu.VMEM((CHUNK,), jnp.float32),
            pltpu.VMEM((CHUNK,), jnp.float32),
            pltpu.VMEM((OFF_PAD,), jnp.int32),
            pltpu.VMEM((LANES,), jnp.int32),
            pltpu.VMEM((LANES,), jnp.float32),
            pltpu.VMEM((BAGS_PER_SUB,), jnp.float32),
            pltpu.VMEM_SHARED((ACC_PAD,), jnp.float32),
            pltpu.SemaphoreType.DMA,
        ],
        compiler_params=pltpu.CompilerParams(needs_layout_passes=False),
    )(ids, offsets, pt0, pt1)


@jax.jit
def kernel(inputs_ids, offsets, emb_weight, fc_weight, fc_bias):
    pt = _project(emb_weight, fc_weight)
    means = _bag_means(inputs_ids.astype(jnp.int32), offsets, pt[0], pt[1])
    return means.T + fc_bias[None, :]


# speedup vs baseline: 731.0468x; 1.0079x over previous
import functools

import jax
import jax.numpy as jnp
from jax import lax
from jax.experimental import pallas as pl
from jax.experimental.pallas import tpu as pltpu
from jax.experimental.pallas import tpu_sc as plsc

VOCAB = 100000
EMBED = 32
NUM_CLASS = 2
BATCH = 16384
TOTAL = 819200

NCORES = 2
NSUB = 16
LANES = 16

TOK_PER_SUB = TOTAL // NSUB
CHUNK = 5120
NCHUNKS = TOK_PER_SUB // CHUNK
GROUPS = CHUNK // LANES
GBATCH = 16
OFF_PAD = BATCH + 2 * LANES
ACC_PAD = BATCH + LANES
BAGS_PER_SUB = BATCH // NSUB


def _proj_body(fc_ref, emb_ref, out_ref):
    out_ref[...] = lax.dot_general(
        fc_ref[...], emb_ref[...],
        dimension_numbers=(((1,), (1,)), ((), ())),
        preferred_element_type=jnp.float32,
    )


def _project(emb_weight, fc_weight):
    VB = 8192
    grid = (VOCAB + VB - 1) // VB
    return pl.pallas_call(
        _proj_body,
        grid=(grid,),
        in_specs=[
            pl.BlockSpec((NUM_CLASS, EMBED), lambda i: (0, 0)),
            pl.BlockSpec((VB, EMBED), lambda i: (i, 0)),
        ],
        out_specs=pl.BlockSpec((NUM_CLASS, VB), lambda i: (0, i)),
        out_shape=jax.ShapeDtypeStruct((NUM_CLASS, VOCAB), jnp.float32),
    )(fc_weight, emb_weight)


def _count_le(off_ref, p_vec):
    r = jnp.zeros((LANES,), jnp.int32)
    step = BATCH // 2
    while step >= 1:
        v = plsc.load_gather(off_ref, [r + (step - 1)])
        r = jnp.where(v <= p_vec, r + step, r)
        step //= 2
    return r


def _bag_body(ids_hbm, off_hbm, pt0_hbm, pt1_hbm, out_hbm,
              ids_v, vals_v, pinc_v, off_v, ibuf_v, cbuf_v, bags_v, acc_sh,
              sem):
    cid = lax.axis_index("c")
    sid = lax.axis_index("s")

    pltpu.sync_copy(off_hbm, off_v.at[pl.ds(0, BATCH)])
    for j in range((OFF_PAD - BATCH) // LANES):
        off_v[pl.ds(BATCH + j * LANES, LANES)] = jnp.full(
            (LANES,), TOTAL, dtype=jnp.int32)

    @pl.when(sid == 0)
    def _zero():
        def zloop(g, _):
            pinc_v[pl.ds(g * LANES, LANES)] = jnp.zeros((LANES,), jnp.float32)
            return 0
        lax.fori_loop(0, GROUPS, zloop, 0)
        for piece in range(3):
            pltpu.sync_copy(pinc_v, acc_sh.at[pl.ds(piece * CHUNK, CHUNK)])
        pltpu.sync_copy(pinc_v.at[pl.ds(0, ACC_PAD - 3 * CHUNK)],
                        acc_sh.at[pl.ds(3 * CHUNK, ACC_PAD - 3 * CHUNK)])

    plsc.subcore_barrier()

    def chunk_body(ci, _):
        tok0 = sid * TOK_PER_SUB + ci * CHUNK

        pltpu.sync_copy(ids_hbm.at[pl.ds(tok0, CHUNK)], ids_v)

        @pl.when(cid == 0)
        def _g0():
            pltpu.async_copy(pt0_hbm.at[ids_v], vals_v, sem).wait()

        @pl.when(cid == 1)
        def _g1():
            pltpu.async_copy(pt1_hbm.at[ids_v], vals_v, sem).wait()

        def cs_body(g, carry):
            v = vals_v[pl.ds(g * LANES, LANES)]
            pinc_v[pl.ds(g * LANES, LANES)] = plsc.cumsum(v) + carry
            return carry + jnp.sum(v)
        lax.fori_loop(0, GROUPS, cs_body, jnp.float32(0.0))

        lane = lax.iota(jnp.int32, LANES)
        p_vec = jnp.where(lane == 0, tok0, tok0 + CHUNK - 1)
        r_vec = _count_le(off_v, p_vec) - 1
        b_lo = r_vec[0]
        b_hi = r_vec[1]
        ngroups = (b_hi - b_lo) // LANES + 1
        nbatches = (ngroups + GBATCH - 1) // GBATCH

        def batch_body(bi, _):
            for gg in range(GBATCH):
                bv = b_lo + (bi * GBATCH + gg) * LANES + lane
                bvc = jnp.minimum(bv, BATCH)
                s16 = plsc.load_gather(off_v, [bvc])
                e16 = plsc.load_gather(off_v, [bvc + 1])
                sc = jnp.clip(s16, tok0, tok0 + CHUNK)
                ec = jnp.clip(e16, tok0, tok0 + CHUNK)
                ilo = sc - tok0 - 1
                ihi = ec - tok0 - 1
                plo = jnp.where(
                    ilo < 0, jnp.float32(0.0),
                    plsc.load_gather(pinc_v, [jnp.maximum(ilo, 0)]))
                phi = jnp.where(
                    ihi < 0, jnp.float32(0.0),
                    plsc.load_gather(pinc_v, [jnp.maximum(ihi, 0)]))
                ibuf_v[pl.ds(gg * LANES, LANES)] = bvc
                cbuf_v[pl.ds(gg * LANES, LANES)] = phi - plo
            pltpu.sync_copy(cbuf_v, acc_sh.at[ibuf_v], add=True)
            return 0
        lax.fori_loop(0, nbatches, batch_body, 0)
        return 0

    lax.fori_loop(0, NCHUNKS, chunk_body, 0)

    plsc.subcore_barrier()

    b0 = sid * BAGS_PER_SUB
    pltpu.sync_copy(acc_sh.at[pl.ds(b0, BAGS_PER_SUB)], bags_v)

    def fin_body(g, _):
        bv = b0 + g * LANES + lax.iota(jnp.int32, LANES)
        s16 = plsc.load_gather(off_v, [bv])
        e16 = plsc.load_gather(off_v, [bv + 1])
        cnt = jnp.maximum((e16 - s16).astype(jnp.float32), 1.0)
        bags_v[pl.ds(g * LANES, LANES)] = (
            bags_v[pl.ds(g * LANES, LANES)] / cnt)
        return 0
    lax.fori_loop(0, BAGS_PER_SUB // LANES, fin_body, 0)

    @pl.when(cid == 0)
    def _w0():
        pltpu.sync_copy(bags_v, out_hbm.at[0, pl.ds(b0, BAGS_PER_SUB)])

    @pl.when(cid == 1)
    def _w1():
        pltpu.sync_copy(bags_v, out_hbm.at[1, pl.ds(b0, BAGS_PER_SUB)])


def _bag_means(ids, offsets, pt0, pt1):
    mesh = plsc.VectorSubcoreMesh(core_axis_name="c", subcore_axis_name="s")
    return pl.kernel(
        _bag_body,
        out_type=jax.ShapeDtypeStruct((NUM_CLASS, BATCH), jnp.float32),
        mesh=mesh,
        scratch_types=[
            pltpu.VMEM((CHUNK,), jnp.int32),
            pltpu.VMEM((CHUNK,), jnp.float32),
            pltpu.VMEM((CHUNK,), jnp.float32),
            pltpu.VMEM((OFF_PAD,), jnp.int32),
            pltpu.VMEM((GBATCH * LANES,), jnp.int32),
            pltpu.VMEM((GBATCH * LANES,), jnp.float32),
            pltpu.VMEM((BAGS_PER_SUB,), jnp.float32),
            pltpu.VMEM_SHARED((ACC_PAD,), jnp.float32),
            pltpu.SemaphoreType.DMA,
        ],
        compiler_params=pltpu.CompilerParams(needs_layout_passes=False),
    )(ids, offsets, pt0, pt1)


@jax.jit
def kernel(inputs_ids, offsets, emb_weight, fc_weight, fc_bias):
    pt = _project(emb_weight, fc_weight)
    means = _bag_means(inputs_ids.astype(jnp.int32), offsets, pt[0], pt[1])
    return means.T + fc_bias[None, :]


# speedup vs baseline: 758.4440x; 1.0375x over previous
import functools

import jax
import jax.numpy as jnp
from jax import lax
from jax.experimental import pallas as pl
from jax.experimental.pallas import tpu as pltpu
from jax.experimental.pallas import tpu_sc as plsc

VOCAB = 100000
EMBED = 32
NUM_CLASS = 2
BATCH = 16384
TOTAL = 819200

NCORES = 2
NSUB = 16
LANES = 16

TOK_PER_SUB = TOTAL // NSUB
CHUNK = 5120
NCHUNKS = TOK_PER_SUB // CHUNK
GROUPS = CHUNK // LANES
GBATCH = 16
OFF_PAD = BATCH + 2 * LANES
ACC_PAD = BATCH + LANES
BAGS_PER_SUB = BATCH // NSUB


def _proj_body(fc_ref, emb_ref, out_ref):
    out_ref[...] = lax.dot_general(
        fc_ref[...], emb_ref[...],
        dimension_numbers=(((1,), (1,)), ((), ())),
        preferred_element_type=jnp.float32,
    )


def _project(emb_weight, fc_weight):
    VB = 8192
    grid = (VOCAB + VB - 1) // VB
    return pl.pallas_call(
        _proj_body,
        grid=(grid,),
        in_specs=[
            pl.BlockSpec((NUM_CLASS, EMBED), lambda i: (0, 0)),
            pl.BlockSpec((VB, EMBED), lambda i: (i, 0)),
        ],
        out_specs=pl.BlockSpec((NUM_CLASS, VB), lambda i: (0, i)),
        out_shape=jax.ShapeDtypeStruct((NUM_CLASS, VOCAB), jnp.float32),
    )(fc_weight, emb_weight)


def _count_le(off_ref, p_vec):
    r = jnp.zeros((LANES,), jnp.int32)
    step = BATCH // 2
    while step >= 1:
        v = plsc.load_gather(off_ref, [r + (step - 1)])
        r = jnp.where(v <= p_vec, r + step, r)
        step //= 2
    return r


def _bag_body(ids_hbm, off_hbm, pt0_hbm, pt1_hbm, out_hbm,
              ids_a, ids_b, vals_a, vals_b, pinc_v, off_v, ibuf_v, cbuf_v,
              bags_v, acc_sh, sem_a, sem_b):
    cid = lax.axis_index("c")
    sid = lax.axis_index("s")

    pltpu.sync_copy(off_hbm, off_v.at[pl.ds(0, BATCH)])
    for j in range((OFF_PAD - BATCH) // LANES):
        off_v[pl.ds(BATCH + j * LANES, LANES)] = jnp.full(
            (LANES,), TOTAL, dtype=jnp.int32)

    @pl.when(sid == 0)
    def _zero():
        def zloop(g, _):
            pinc_v[pl.ds(g * LANES, LANES)] = jnp.zeros((LANES,), jnp.float32)
            return 0
        lax.fori_loop(0, GROUPS, zloop, 0)
        for piece in range(3):
            pltpu.sync_copy(pinc_v, acc_sh.at[pl.ds(piece * CHUNK, CHUNK)])
        pltpu.sync_copy(pinc_v.at[pl.ds(0, ACC_PAD - 3 * CHUNK)],
                        acc_sh.at[pl.ds(3 * CHUNK, ACC_PAD - 3 * CHUNK)])

    plsc.subcore_barrier()

    bufs = ((ids_a, vals_a, sem_a), (ids_b, vals_b, sem_b))

    def _start(ci):
        ids_v, vals_v, sem = bufs[ci % 2]
        tok0 = sid * TOK_PER_SUB + ci * CHUNK
        pltpu.sync_copy(ids_hbm.at[pl.ds(tok0, CHUNK)], ids_v)

        @pl.when(cid == 0)
        def _g0():
            pltpu.async_copy(pt0_hbm.at[ids_v], vals_v, sem)

        @pl.when(cid == 1)
        def _g1():
            pltpu.async_copy(pt1_hbm.at[ids_v], vals_v, sem)

    _start(0)
    for ci in range(NCHUNKS):
        ids_v, vals_v, sem = bufs[ci % 2]
        if ci + 1 < NCHUNKS:
            _start(ci + 1)
        pltpu.make_async_copy(pt0_hbm.at[ids_v], vals_v, sem).wait()
        tok0 = sid * TOK_PER_SUB + ci * CHUNK

        def cs_body(g, carry):
            v = vals_v[pl.ds(g * LANES, LANES)]
            pinc_v[pl.ds(g * LANES, LANES)] = plsc.cumsum(v) + carry
            return carry + jnp.sum(v)
        lax.fori_loop(0, GROUPS, cs_body, jnp.float32(0.0))

        lane = lax.iota(jnp.int32, LANES)
        p_vec = jnp.where(lane == 0, tok0, tok0 + CHUNK - 1)
        r_vec = _count_le(off_v, p_vec) - 1
        b_lo = r_vec[0]
        b_hi = r_vec[1]
        ngroups = (b_hi - b_lo) // LANES + 1
        nbatches = (ngroups + GBATCH - 1) // GBATCH

        def batch_body(bi, _):
            for gg in range(GBATCH):
                bv = b_lo + (bi * GBATCH + gg) * LANES + lane
                bvc = jnp.minimum(bv, BATCH)
                s16 = plsc.load_gather(off_v, [bvc])
                e16 = plsc.load_gather(off_v, [bvc + 1])
                sc = jnp.clip(s16, tok0, tok0 + CHUNK)
                ec = jnp.clip(e16, tok0, tok0 + CHUNK)
                ilo = sc - tok0 - 1
                ihi = ec - tok0 - 1
                plo = jnp.where(
                    ilo < 0, jnp.float32(0.0),
                    plsc.load_gather(pinc_v, [jnp.maximum(ilo, 0)]))
                phi = jnp.where(
                    ihi < 0, jnp.float32(0.0),
                    plsc.load_gather(pinc_v, [jnp.maximum(ihi, 0)]))
                ibuf_v[pl.ds(gg * LANES, LANES)] = bvc
                cbuf_v[pl.ds(gg * LANES, LANES)] = phi - plo
            pltpu.sync_copy(cbuf_v, acc_sh.at[ibuf_v], add=True)
            return 0
        lax.fori_loop(0, nbatches, batch_body, 0)

    plsc.subcore_barrier()

    b0 = sid * BAGS_PER_SUB
    pltpu.sync_copy(acc_sh.at[pl.ds(b0, BAGS_PER_SUB)], bags_v)

    def fin_body(g, _):
        bv = b0 + g * LANES + lax.iota(jnp.int32, LANES)
        s16 = plsc.load_gather(off_v, [bv])
        e16 = plsc.load_gather(off_v, [bv + 1])
        cnt = jnp.maximum((e16 - s16).astype(jnp.float32), 1.0)
        bags_v[pl.ds(g * LANES, LANES)] = (
            bags_v[pl.ds(g * LANES, LANES)] / cnt)
        return 0
    lax.fori_loop(0, BAGS_PER_SUB // LANES, fin_body, 0)

    @pl.when(cid == 0)
    def _w0():
        pltpu.sync_copy(bags_v, out_hbm.at[0, pl.ds(b0, BAGS_PER_SUB)])

    @pl.when(cid == 1)
    def _w1():
        pltpu.sync_copy(bags_v, out_hbm.at[1, pl.ds(b0, BAGS_PER_SUB)])


def _bag_means(ids, offsets, pt0, pt1):
    mesh = plsc.VectorSubcoreMesh(core_axis_name="c", subcore_axis_name="s")
    return pl.kernel(
        _bag_body,
        out_type=jax.ShapeDtypeStruct((NUM_CLASS, BATCH), jnp.float32),
        mesh=mesh,
        scratch_types=[
            pltpu.VMEM((CHUNK,), jnp.int32),
            pltpu.VMEM((CHUNK,), jnp.int32),
            pltpu.VMEM((CHUNK,), jnp.float32),
            pltpu.VMEM((CHUNK,), jnp.float32),
            pltpu.VMEM((CHUNK,), jnp.float32),
            pltpu.VMEM((OFF_PAD,), jnp.int32),
            pltpu.VMEM((GBATCH * LANES,), jnp.int32),
            pltpu.VMEM((GBATCH * LANES,), jnp.float32),
            pltpu.VMEM((BAGS_PER_SUB,), jnp.float32),
            pltpu.VMEM_SHARED((ACC_PAD,), jnp.float32),
            pltpu.SemaphoreType.DMA,
            pltpu.SemaphoreType.DMA,
        ],
        compiler_params=pltpu.CompilerParams(needs_layout_passes=False),
    )(ids, offsets, pt0, pt1)


@jax.jit
def kernel(inputs_ids, offsets, emb_weight, fc_weight, fc_bias):
    pt = _project(emb_weight, fc_weight)
    means = _bag_means(inputs_ids.astype(jnp.int32), offsets, pt[0], pt[1])
    return means.T + fc_bias[None, :]


# speedup vs baseline: 758.9625x; 1.0007x over previous
import functools

import jax
import jax.numpy as jnp
from jax import lax
from jax.experimental import pallas as pl
from jax.experimental.pallas import tpu as pltpu
from jax.experimental.pallas import tpu_sc as plsc

VOCAB = 100000
EMBED = 32
NUM_CLASS = 2
BATCH = 16384
TOTAL = 819200

NCORES = 2
NSUB = 16
LANES = 16

TOK_PER_SUB = TOTAL // NSUB
CHUNK = 5120
NCHUNKS = TOK_PER_SUB // CHUNK
GROUPS = CHUNK // LANES
GBATCH = 16
CS_UNROLL = 8
OFF_PAD = BATCH + 2 * LANES
ACC_PAD = BATCH + LANES
BAGS_PER_SUB = BATCH // NSUB


def _proj_body(fc_ref, emb_ref, out_ref):
    out_ref[...] = lax.dot_general(
        fc_ref[...], emb_ref[...],
        dimension_numbers=(((1,), (1,)), ((), ())),
        preferred_element_type=jnp.float32,
    )


def _project(emb_weight, fc_weight):
    VB = 8192
    grid = (VOCAB + VB - 1) // VB
    return pl.pallas_call(
        _proj_body,
        grid=(grid,),
        in_specs=[
            pl.BlockSpec((NUM_CLASS, EMBED), lambda i: (0, 0)),
            pl.BlockSpec((VB, EMBED), lambda i: (i, 0)),
        ],
        out_specs=pl.BlockSpec((NUM_CLASS, VB), lambda i: (0, i)),
        out_shape=jax.ShapeDtypeStruct((NUM_CLASS, VOCAB), jnp.float32),
    )(fc_weight, emb_weight)


def _count_le(off_ref, p_vec):
    r = jnp.zeros((LANES,), jnp.int32)
    step = BATCH // 2
    while step >= 1:
        v = plsc.load_gather(off_ref, [r + (step - 1)])
        r = jnp.where(v <= p_vec, r + step, r)
        step //= 2
    return r


def _bag_body(ids_hbm, off_hbm, pt0_hbm, pt1_hbm, out_hbm,
              ids_a, ids_b, vals_a, vals_b, pinc_v, off_v, ibuf_v, cbuf_v,
              bags_v, acc_sh, sem_a, sem_b):
    cid = lax.axis_index("c")
    sid = lax.axis_index("s")

    pltpu.sync_copy(off_hbm, off_v.at[pl.ds(0, BATCH)])
    for j in range((OFF_PAD - BATCH) // LANES):
        off_v[pl.ds(BATCH + j * LANES, LANES)] = jnp.full(
            (LANES,), TOTAL, dtype=jnp.int32)

    @pl.when(sid == 0)
    def _zero():
        def zloop(g, _):
            pinc_v[pl.ds(g * LANES, LANES)] = jnp.zeros((LANES,), jnp.float32)
            return 0
        lax.fori_loop(0, GROUPS, zloop, 0)
        for piece in range(3):
            pltpu.sync_copy(pinc_v, acc_sh.at[pl.ds(piece * CHUNK, CHUNK)])
        pltpu.sync_copy(pinc_v.at[pl.ds(0, ACC_PAD - 3 * CHUNK)],
                        acc_sh.at[pl.ds(3 * CHUNK, ACC_PAD - 3 * CHUNK)])

    plsc.subcore_barrier()

    bufs = ((ids_a, vals_a, sem_a), (ids_b, vals_b, sem_b))

    def _start(ci):
        ids_v, vals_v, sem = bufs[ci % 2]
        tok0 = sid * TOK_PER_SUB + ci * CHUNK
        pltpu.sync_copy(ids_hbm.at[pl.ds(tok0, CHUNK)], ids_v)

        @pl.when(cid == 0)
        def _g0():
            pltpu.async_copy(pt0_hbm.at[ids_v], vals_v, sem)

        @pl.when(cid == 1)
        def _g1():
            pltpu.async_copy(pt1_hbm.at[ids_v], vals_v, sem)

    _start(0)
    for ci in range(NCHUNKS):
        ids_v, vals_v, sem = bufs[ci % 2]
        if ci + 1 < NCHUNKS:
            _start(ci + 1)
        pltpu.make_async_copy(pt0_hbm.at[ids_v], vals_v, sem).wait()
        tok0 = sid * TOK_PER_SUB + ci * CHUNK

        def cs_body(g, carry):
            for u in range(CS_UNROLL):
                off = (g * CS_UNROLL + u) * LANES
                v = vals_v[pl.ds(off, LANES)]
                pinc_v[pl.ds(off, LANES)] = plsc.cumsum(v) + carry
                carry = carry + jnp.sum(v)
            return carry
        lax.fori_loop(0, GROUPS // CS_UNROLL, cs_body, jnp.float32(0.0))

        lane = lax.iota(jnp.int32, LANES)
        p_vec = jnp.where(lane == 0, tok0, tok0 + CHUNK - 1)
        r_vec = _count_le(off_v, p_vec) - 1
        b_lo = r_vec[0]
        b_hi = r_vec[1]
        ngroups = (b_hi - b_lo) // LANES + 1
        nbatches = (ngroups + GBATCH - 1) // GBATCH

        def batch_body(bi, _):
            for gg in range(GBATCH):
                bv = b_lo + (bi * GBATCH + gg) * LANES + lane
                bvc = jnp.minimum(bv, BATCH)
                s16 = plsc.load_gather(off_v, [bvc])
                e16 = plsc.load_gather(off_v, [bvc + 1])
                sc = jnp.clip(s16, tok0, tok0 + CHUNK)
                ec = jnp.clip(e16, tok0, tok0 + CHUNK)
                ilo = sc - tok0 - 1
                ihi = ec - tok0 - 1
                plo = jnp.where(
                    ilo < 0, jnp.float32(0.0),
                    plsc.load_gather(pinc_v, [jnp.maximum(ilo, 0)]))
                phi = jnp.where(
                    ihi < 0, jnp.float32(0.0),
                    plsc.load_gather(pinc_v, [jnp.maximum(ihi, 0)]))
                ibuf_v[pl.ds(gg * LANES, LANES)] = bvc
                cbuf_v[pl.ds(gg * LANES, LANES)] = phi - plo
            pltpu.sync_copy(cbuf_v, acc_sh.at[ibuf_v], add=True)
            return 0
        lax.fori_loop(0, nbatches, batch_body, 0)

    plsc.subcore_barrier()

    b0 = sid * BAGS_PER_SUB
    pltpu.sync_copy(acc_sh.at[pl.ds(b0, BAGS_PER_SUB)], bags_v)

    def fin_body(g, _):
        bv = b0 + g * LANES + lax.iota(jnp.int32, LANES)
        s16 = plsc.load_gather(off_v, [bv])
        e16 = plsc.load_gather(off_v, [bv + 1])
        cnt = jnp.maximum((e16 - s16).astype(jnp.float32), 1.0)
        bags_v[pl.ds(g * LANES, LANES)] = (
            bags_v[pl.ds(g * LANES, LANES)] / cnt)
        return 0
    lax.fori_loop(0, BAGS_PER_SUB // LANES, fin_body, 0)

    @pl.when(cid == 0)
    def _w0():
        pltpu.sync_copy(bags_v, out_hbm.at[0, pl.ds(b0, BAGS_PER_SUB)])

    @pl.when(cid == 1)
    def _w1():
        pltpu.sync_copy(bags_v, out_hbm.at[1, pl.ds(b0, BAGS_PER_SUB)])


def _bag_means(ids, offsets, pt0, pt1):
    mesh = plsc.VectorSubcoreMesh(core_axis_name="c", subcore_axis_name="s")
    return pl.kernel(
        _bag_body,
        out_type=jax.ShapeDtypeStruct((NUM_CLASS, BATCH), jnp.float32),
        mesh=mesh,
        scratch_types=[
            pltpu.VMEM((CHUNK,), jnp.int32),
            pltpu.VMEM((CHUNK,), jnp.int32),
            pltpu.VMEM((CHUNK,), jnp.float32),
            pltpu.VMEM((CHUNK,), jnp.float32),
            pltpu.VMEM((CHUNK,), jnp.float32),
            pltpu.VMEM((OFF_PAD,), jnp.int32),
            pltpu.VMEM((GBATCH * LANES,), jnp.int32),
            pltpu.VMEM((GBATCH * LANES,), jnp.float32),
            pltpu.VMEM((BAGS_PER_SUB,), jnp.float32),
            pltpu.VMEM_SHARED((ACC_PAD,), jnp.float32),
            pltpu.SemaphoreType.DMA,
            pltpu.SemaphoreType.DMA,
        ],
        compiler_params=pltpu.CompilerParams(needs_layout_passes=False),
    )(ids, offsets, pt0, pt1)


@jax.jit
def kernel(inputs_ids, offsets, emb_weight, fc_weight, fc_bias):
    pt = _project(emb_weight, fc_weight)
    means = _bag_means(inputs_ids.astype(jnp.int32), offsets, pt[0], pt[1])
    return means.T + fc_bias[None, :]
